# Initial kernel scaffold; baseline (speedup 1.0000x reference)
import jax
import jax.numpy as jnp
from jax.experimental import pallas as pl


def kernel(x, edge_index, edge_attr, params):
    raise NotImplementedError("write your pallas kernel here")


# speedup vs baseline: 1.6254x; 1.6254x over previous
import functools

import jax
import jax.numpy as jnp
from jax import lax
from jax.experimental import pallas as pl
from jax.experimental.pallas import tpu as pltpu
from jax.experimental.pallas import tpu_sc as plsc

N = 50000
E = 800000
HID = 64

NC = 2
NS = 16
NW = NC * NS

_mesh = plsc.VectorSubcoreMesh(core_axis_name="c", subcore_axis_name="s")

CH = 128
EPW = 24960
MAIN = EPW * NW
TAIL = (E - MAIN) // NW
TW = 2 * HID


@functools.partial(
    pl.kernel,
    out_type=jax.ShapeDtypeStruct((E, HID), jnp.float32),
    mesh=_mesh,
    scratch_types=(
        pltpu.VMEM((CH,), jnp.int32),
        pltpu.VMEM((CH,), jnp.int32),
        pltpu.VMEM((CH, TW), jnp.float32),
        pltpu.VMEM((CH, TW), jnp.float32),
        pltpu.VMEM((CH, HID), jnp.float32),
        pltpu.SemaphoreType.DMA,
        pltpu.SemaphoreType.DMA,
    ),
    compiler_params=pltpu.CompilerParams(use_tc_tiling_on_sc=False),
)
def _sc_gather_add(t_hbm, row_hbm, col_hbm, g_out,
                   ia, ib, ba, bb, gv, sa, sb):
    wid = lax.axis_index("s") * NC + lax.axis_index("c")
    base = wid * EPW

    def chunk(off, n):
        pltpu.sync_copy(row_hbm.at[pl.ds(off, n)], ia.at[pl.ds(0, n)])
        pltpu.sync_copy(col_hbm.at[pl.ds(off, n)], ib.at[pl.ds(0, n)])
        ca = pltpu.async_copy(t_hbm.at[ia.at[pl.ds(0, n)]],
                              ba.at[pl.ds(0, n)], sa)
        cb = pltpu.async_copy(t_hbm.at[ib.at[pl.ds(0, n)]],
                              bb.at[pl.ds(0, n)], sb)
        ca.wait()
        cb.wait()

        def addrow(k, carry):
            for j in range(HID // 16):
                gv[k, pl.ds(j * 16, 16)] = (
                    ba[k, pl.ds(j * 16, 16)]
                    + bb[k, pl.ds(HID + j * 16, 16)])
            return carry

        lax.fori_loop(0, n, addrow, 0)
        pltpu.sync_copy(gv.at[pl.ds(0, n)], g_out.at[pl.ds(off, n)])

    def body(i, carry):
        chunk(base + i * CH, CH)
        return carry

    lax.fori_loop(0, EPW // CH, body, 0)
    chunk(MAIN + wid * TAIL, TAIL)


SCH = 128
EPT = 49920
STAIL = (E - EPT * NS) // NS
HALF = N // NC
ZROWS = 1568
ACC_ROWS = ZROWS * NS
TRASH = 25024
LASTZ = HALF - (NS - 1) * ZROWS


@functools.partial(
    pl.kernel,
    out_type=jax.ShapeDtypeStruct((N, HID), jnp.float32),
    mesh=_mesh,
    scratch_types=(
        pltpu.VMEM((SCH,), jnp.int32),
        pltpu.VMEM((SCH, HID), jnp.float32),
        pltpu.VMEM((STAIL,), jnp.int32),
        pltpu.VMEM((STAIL, HID), jnp.float32),
        pltpu.VMEM_SHARED((ACC_ROWS, HID), jnp.float32),
    ),
    compiler_params=pltpu.CompilerParams(use_tc_tiling_on_sc=False),
)
def _sc_scatter(loc_hbm, val_hbm, zeros_hbm, out_hbm,
                il, vv, ilt, vvt, acc):
    cid = lax.axis_index("c")
    tid = lax.axis_index("s")
    nbase = cid * HALF
    ebase = cid * E

    pltpu.sync_copy(zeros_hbm, acc.at[pl.ds(tid * ZROWS, ZROWS)])
    plsc.subcore_barrier()

    def chunk(off, i_l, v_v, n):
        pltpu.sync_copy(loc_hbm.at[pl.ds(ebase + off, n)], i_l)
        pltpu.sync_copy(val_hbm.at[pl.ds(off, n)], v_v)
        pltpu.sync_copy(v_v, acc.at[i_l], add=True)

    def body(i, carry):
        chunk(tid * EPT + i * SCH, il, vv, SCH)
        return carry

    lax.fori_loop(0, EPT // SCH, body, 0)
    chunk(EPT * NS + tid * STAIL, ilt, vvt, STAIL)
    plsc.subcore_barrier()

    @pl.when(tid < NS - 1)
    def _():
        pltpu.sync_copy(acc.at[pl.ds(tid * ZROWS, ZROWS)],
                        out_hbm.at[pl.ds(nbase + tid * ZROWS, ZROWS)])

    @pl.when(tid == NS - 1)
    def _():
        pltpu.sync_copy(acc.at[pl.ds((NS - 1) * ZROWS, LASTZ)],
                        out_hbm.at[pl.ds(nbase + (NS - 1) * ZROWS, LASTZ)])


CNTW = 16


@functools.partial(
    pl.kernel,
    out_type=jax.ShapeDtypeStruct((N, CNTW), jnp.float32),
    mesh=_mesh,
    scratch_types=(
        pltpu.VMEM((SCH,), jnp.int32),
        pltpu.VMEM((SCH, CNTW), jnp.float32),
        pltpu.VMEM((STAIL,), jnp.int32),
        pltpu.VMEM_SHARED((ACC_ROWS, CNTW), jnp.float32),
    ),
    compiler_params=pltpu.CompilerParams(use_tc_tiling_on_sc=False),
)
def _sc_count(loc_hbm, ones_hbm, zeros_hbm, out_hbm,
              il, ones_v, ilt, acc):
    cid = lax.axis_index("c")
    tid = lax.axis_index("s")
    nbase = cid * HALF
    ebase = cid * E

    pltpu.sync_copy(zeros_hbm, acc.at[pl.ds(tid * ZROWS, ZROWS)])
    pltpu.sync_copy(ones_hbm, ones_v)
    plsc.subcore_barrier()

    def chunk(off, i_l, n):
        pltpu.sync_copy(loc_hbm.at[pl.ds(ebase + off, n)], i_l)
        pltpu.sync_copy(ones_v.at[pl.ds(0, n)], acc.at[i_l], add=True)

    def body(i, carry):
        chunk(tid * EPT + i * SCH, il, SCH)
        return carry

    lax.fori_loop(0, EPT // SCH, body, 0)
    chunk(EPT * NS + tid * STAIL, ilt, STAIL)
    plsc.subcore_barrier()

    @pl.when(tid < NS - 1)
    def _():
        pltpu.sync_copy(acc.at[pl.ds(tid * ZROWS, ZROWS)],
                        out_hbm.at[pl.ds(nbase + tid * ZROWS, ZROWS)])

    @pl.when(tid == NS - 1)
    def _():
        pltpu.sync_copy(acc.at[pl.ds((NS - 1) * ZROWS, LASTZ)],
                        out_hbm.at[pl.ds(nbase + (NS - 1) * ZROWS, LASTZ)])


BE = 2000
BN = 2000

_tc_params = pltpu.CompilerParams(dimension_semantics=("arbitrary",))


def _full2(shape):
    return pl.BlockSpec(shape, lambda i: (0, 0))


def _rows(shape):
    return pl.BlockSpec(shape, lambda i: (i, 0))


def _ln(r, g, b):
    m = jnp.mean(r, axis=-1, keepdims=True)
    v = jnp.mean((r - m) ** 2, axis=-1, keepdims=True)
    return (r - m) * lax.rsqrt(v + 1e-5) * g + b


def _pair_body(h_ref, wa, wb, out_ref):
    out_ref[...] = jnp.concatenate(
        [jnp.dot(h_ref[...], wa[...], preferred_element_type=jnp.float32),
         jnp.dot(h_ref[...], wb[...], preferred_element_type=jnp.float32)],
        axis=-1)


def _tc_pair(h, wa, wb):
    return pl.pallas_call(
        _pair_body,
        grid=(N // BN,),
        in_specs=[_rows((BN, HID)), _full2((HID, HID)), _full2((HID, HID))],
        out_specs=_rows((BN, TW)),
        out_shape=jax.ShapeDtypeStruct((N, TW), jnp.float32),
        compiler_params=_tc_params,
    )(h, wa, wb)


def _edge_body(g_ref, e_ref, wc, b1, g, bt, w2, b2,
               enew_ref, enext_ref):
    pre = (g_ref[...]
           + jnp.dot(e_ref[...], wc[...], preferred_element_type=jnp.float32)
           + b1[...])
    r = jnp.maximum(pre, 0.0)
    ln = _ln(r, g[...], bt[...])
    en = jnp.dot(ln, w2[...], preferred_element_type=jnp.float32) + b2[...]
    enew_ref[...] = en
    enext_ref[...] = e_ref[...] + en


def _tc_edge(gsum, e, wc, b1, g, bt, w2, b2):
    return pl.pallas_call(
        _edge_body,
        grid=(E // BE,),
        in_specs=[_rows((BE, HID)), _rows((BE, HID)),
                  _full2((HID, HID)),
                  _full2((1, HID)), _full2((1, HID)), _full2((1, HID)),
                  _full2((HID, HID)), _full2((1, HID))],
        out_specs=[_rows((BE, HID)), _rows((BE, HID))],
        out_shape=(jax.ShapeDtypeStruct((E, HID), jnp.float32),
                   jax.ShapeDtypeStruct((E, HID), jnp.float32)),
        compiler_params=_tc_params,
    )(gsum, e, wc, b1.reshape(1, HID), g.reshape(1, HID),
      bt.reshape(1, HID), w2, b2.reshape(1, HID))


def _node_body(h_ref, s_ref, cnt_ref, wh, wa, b1, g, bt, w2, b2, out_ref):
    inv = 1.0 / jnp.maximum(cnt_ref[...][:, :1], 1.0)
    agg = s_ref[...] * inv
    pre = (jnp.dot(h_ref[...], wh[...], preferred_element_type=jnp.float32)
           + jnp.dot(agg, wa[...], preferred_element_type=jnp.float32)
           + b1[...])
    r = jnp.maximum(pre, 0.0)
    ln = _ln(r, g[...], bt[...])
    out_ref[...] = h_ref[...] + jnp.dot(
        ln, w2[...], preferred_element_type=jnp.float32) + b2[...]


def _tc_node(h, s, cnt, wh, wa, b1, g, bt, w2, b2):
    return pl.pallas_call(
        _node_body,
        grid=(N // BN,),
        in_specs=[_rows((BN, HID)), _rows((BN, HID)), _rows((BN, CNTW)),
                  _full2((HID, HID)), _full2((HID, HID)),
                  _full2((1, HID)), _full2((1, HID)), _full2((1, HID)),
                  _full2((HID, HID)), _full2((1, HID))],
        out_specs=_rows((BN, HID)),
        out_shape=jax.ShapeDtypeStruct((N, HID), jnp.float32),
        compiler_params=_tc_params,
    )(h, s, cnt, wh, wa, b1.reshape(1, HID), g.reshape(1, HID),
      bt.reshape(1, HID), w2, b2.reshape(1, HID))


def _enc_body(x_ref, w1, b1, w2, b2, out_ref):
    r = jnp.maximum(
        jnp.dot(x_ref[...], w1[...], preferred_element_type=jnp.float32)
        + b1[...], 0.0)
    out_ref[...] = jnp.dot(r, w2[...], preferred_element_type=jnp.float32) + b2[...]


def _tc_enc(x, w1, b1, w2, b2, blk_rows):
    rows, din = x.shape
    dout = w2.shape[1]
    return pl.pallas_call(
        _enc_body,
        grid=(rows // blk_rows,),
        in_specs=[_rows((blk_rows, din)), _full2((din, HID)), _full2((1, HID)),
                  _full2((HID, dout)), _full2((1, dout))],
        out_specs=_rows((blk_rows, dout)),
        out_shape=jax.ShapeDtypeStruct((rows, dout), jnp.float32),
        compiler_params=_tc_params,
    )(x, w1, b1.reshape(1, -1), w2, b2.reshape(1, -1))


def kernel(x, edge_index, edge_attr, params):
    row = edge_index[0]
    col = edge_index[1]

    loc0 = jnp.where(col < HALF, col, TRASH)
    loc1 = jnp.where(col >= HALF, col - HALF, TRASH)
    loc01 = jnp.concatenate([loc0, loc1]).astype(jnp.int32)

    ones = jnp.ones((SCH, CNTW), jnp.float32)
    zeros16 = jnp.zeros((ZROWS, CNTW), jnp.float32)
    zeros64 = jnp.zeros((ZROWS, HID), jnp.float32)
    cnt = _sc_count(loc01, ones, zeros16)

    dep = jnp.minimum(cnt[0, 0], 0.0)

    (wn1, bn1), (wn2, bn2) = params['enc_n']
    h = _tc_enc(x, wn1 + dep, bn1, wn2, bn2, BN)
    (we1, be1), (we2, be2) = params['enc_e']
    e = _tc_enc(edge_attr, we1 + dep, be1, we2, be2, BE)

    for blk in params['blocks']:
        ew1, eb1 = blk['edge']['l1']
        ew2, eb2 = blk['edge']['l2']
        nw1, nb1 = blk['node']['l1']
        nw2, nb2 = blk['node']['l2']

        t = _tc_pair(h, ew1[:HID], ew1[HID:2 * HID])
        gsum = _sc_gather_add(t, row, col)
        e_new, e = _tc_edge(gsum, e, ew1[2 * HID:],
                            eb1, blk['edge']['ln_g'], blk['edge']['ln_b'],
                            ew2, eb2)
        s = _sc_scatter(loc01, e_new, zeros64)
        h = _tc_node(h, s, cnt, nw1[:HID], nw1[HID:], nb1,
                     blk['node']['ln_g'], blk['node']['ln_b'], nw2, nb2)

    (dw1, db1), (dw2, db2) = params['dec']
    return _tc_enc(h, dw1, db1, dw2, db2, BN)


# speedup vs baseline: 1.8914x; 1.1636x over previous
import functools

import jax
import jax.numpy as jnp
from jax import lax
from jax.experimental import pallas as pl
from jax.experimental.pallas import tpu as pltpu
from jax.experimental.pallas import tpu_sc as plsc

N = 50000
E = 800000
HID = 64

NC = 2
NS = 16
NW = NC * NS

_mesh = plsc.VectorSubcoreMesh(core_axis_name="c", subcore_axis_name="s")

CH = 128
EPW = 24960
MAIN = EPW * NW
TAIL = (E - MAIN) // NW
TW = 2 * HID


@functools.partial(
    pl.kernel,
    out_type=jax.ShapeDtypeStruct((E, HID), jnp.float32),
    mesh=_mesh,
    scratch_types=(
        pltpu.VMEM((CH,), jnp.int32),
        pltpu.VMEM((CH,), jnp.int32),
        pltpu.VMEM((CH, HID), jnp.float32),
        pltpu.VMEM((CH, HID), jnp.float32),
        pltpu.VMEM((CH, HID), jnp.float32),
        pltpu.SemaphoreType.DMA,
        pltpu.SemaphoreType.DMA,
    ),
    compiler_params=pltpu.CompilerParams(use_tc_tiling_on_sc=False),
)
def _sc_gather_add(ts_hbm, td_hbm, row_hbm, col_hbm, g_out,
                   ia, ib, ba, bb, gv, sa, sb):
    wid = lax.axis_index("s") * NC + lax.axis_index("c")
    base = wid * EPW

    def chunk(off, n):
        pltpu.sync_copy(row_hbm.at[pl.ds(off, n)], ia.at[pl.ds(0, n)])
        pltpu.sync_copy(col_hbm.at[pl.ds(off, n)], ib.at[pl.ds(0, n)])
        ca = pltpu.async_copy(ts_hbm.at[ia.at[pl.ds(0, n)]],
                              ba.at[pl.ds(0, n)], sa)
        cb = pltpu.async_copy(td_hbm.at[ib.at[pl.ds(0, n)]],
                              bb.at[pl.ds(0, n)], sb)
        ca.wait()
        cb.wait()

        def addrow(k, carry):
            for j in range(HID // 16):
                gv[k, pl.ds(j * 16, 16)] = (
                    ba[k, pl.ds(j * 16, 16)]
                    + bb[k, pl.ds(j * 16, 16)])
            return carry

        lax.fori_loop(0, n, addrow, 0)
        pltpu.sync_copy(gv.at[pl.ds(0, n)], g_out.at[pl.ds(off, n)])

    def body(i, carry):
        chunk(base + i * CH, CH)
        return carry

    lax.fori_loop(0, EPW // CH, body, 0)
    chunk(MAIN + wid * TAIL, TAIL)


SCH = 128
EPT = 49920
STAIL = (E - EPT * NS) // NS
HALF = N // NC
ZROWS = 1568
ACC_ROWS = ZROWS * NS
TRASH = 25024
LASTZ = HALF - (NS - 1) * ZROWS


@functools.partial(
    pl.kernel,
    out_type=jax.ShapeDtypeStruct((N, HID), jnp.float32),
    mesh=_mesh,
    scratch_types=(
        pltpu.VMEM((SCH,), jnp.int32),
        pltpu.VMEM((SCH, HID), jnp.float32),
        pltpu.VMEM((STAIL,), jnp.int32),
        pltpu.VMEM((STAIL, HID), jnp.float32),
        pltpu.VMEM_SHARED((ACC_ROWS, HID), jnp.float32),
    ),
    compiler_params=pltpu.CompilerParams(use_tc_tiling_on_sc=False),
)
def _sc_scatter(loc_hbm, val_hbm, zeros_hbm, out_hbm,
                il, vv, ilt, vvt, acc):
    cid = lax.axis_index("c")
    tid = lax.axis_index("s")
    nbase = cid * HALF
    ebase = cid * E

    pltpu.sync_copy(zeros_hbm, acc.at[pl.ds(tid * ZROWS, ZROWS)])
    plsc.subcore_barrier()

    def chunk(off, i_l, v_v, n):
        pltpu.sync_copy(loc_hbm.at[pl.ds(ebase + off, n)], i_l)
        pltpu.sync_copy(val_hbm.at[pl.ds(off, n)], v_v)
        pltpu.sync_copy(v_v, acc.at[i_l], add=True)

    def body(i, carry):
        chunk(tid * EPT + i * SCH, il, vv, SCH)
        return carry

    lax.fori_loop(0, EPT // SCH, body, 0)
    chunk(EPT * NS + tid * STAIL, ilt, vvt, STAIL)
    plsc.subcore_barrier()

    @pl.when(tid < NS - 1)
    def _():
        pltpu.sync_copy(acc.at[pl.ds(tid * ZROWS, ZROWS)],
                        out_hbm.at[pl.ds(nbase + tid * ZROWS, ZROWS)])

    @pl.when(tid == NS - 1)
    def _():
        pltpu.sync_copy(acc.at[pl.ds((NS - 1) * ZROWS, LASTZ)],
                        out_hbm.at[pl.ds(nbase + (NS - 1) * ZROWS, LASTZ)])


CNTW = 16


@functools.partial(
    pl.kernel,
    out_type=jax.ShapeDtypeStruct((N, CNTW), jnp.float32),
    mesh=_mesh,
    scratch_types=(
        pltpu.VMEM((SCH,), jnp.int32),
        pltpu.VMEM((SCH, CNTW), jnp.float32),
        pltpu.VMEM((STAIL,), jnp.int32),
        pltpu.VMEM_SHARED((ACC_ROWS, CNTW), jnp.float32),
    ),
    compiler_params=pltpu.CompilerParams(use_tc_tiling_on_sc=False),
)
def _sc_count(loc_hbm, ones_hbm, zeros_hbm, out_hbm,
              il, ones_v, ilt, acc):
    cid = lax.axis_index("c")
    tid = lax.axis_index("s")
    nbase = cid * HALF
    ebase = cid * E

    pltpu.sync_copy(zeros_hbm, acc.at[pl.ds(tid * ZROWS, ZROWS)])
    pltpu.sync_copy(ones_hbm, ones_v)
    plsc.subcore_barrier()

    def chunk(off, i_l, n):
        pltpu.sync_copy(loc_hbm.at[pl.ds(ebase + off, n)], i_l)
        pltpu.sync_copy(ones_v.at[pl.ds(0, n)], acc.at[i_l], add=True)

    def body(i, carry):
        chunk(tid * EPT + i * SCH, il, SCH)
        return carry

    lax.fori_loop(0, EPT // SCH, body, 0)
    chunk(EPT * NS + tid * STAIL, ilt, STAIL)
    plsc.subcore_barrier()

    @pl.when(tid < NS - 1)
    def _():
        pltpu.sync_copy(acc.at[pl.ds(tid * ZROWS, ZROWS)],
                        out_hbm.at[pl.ds(nbase + tid * ZROWS, ZROWS)])

    @pl.when(tid == NS - 1)
    def _():
        pltpu.sync_copy(acc.at[pl.ds((NS - 1) * ZROWS, LASTZ)],
                        out_hbm.at[pl.ds(nbase + (NS - 1) * ZROWS, LASTZ)])


BE = 2000
BN = 2000

_tc_params = pltpu.CompilerParams(dimension_semantics=("arbitrary",))


def _full2(shape):
    return pl.BlockSpec(shape, lambda i: (0, 0))


def _rows(shape):
    return pl.BlockSpec(shape, lambda i: (i, 0))


def _ln(r, g, b):
    m = jnp.mean(r, axis=-1, keepdims=True)
    v = jnp.mean((r - m) ** 2, axis=-1, keepdims=True)
    return (r - m) * lax.rsqrt(v + 1e-5) * g + b


def _pair_body(h_ref, wa, wb, ts_ref, td_ref):
    ts_ref[...] = jnp.dot(h_ref[...], wa[...], preferred_element_type=jnp.float32)
    td_ref[...] = jnp.dot(h_ref[...], wb[...], preferred_element_type=jnp.float32)


def _tc_pair(h, wa, wb):
    return pl.pallas_call(
        _pair_body,
        grid=(N // BN,),
        in_specs=[_rows((BN, HID)), _full2((HID, HID)), _full2((HID, HID))],
        out_specs=[_rows((BN, HID)), _rows((BN, HID))],
        out_shape=(jax.ShapeDtypeStruct((N, HID), jnp.float32),
                   jax.ShapeDtypeStruct((N, HID), jnp.float32)),
        compiler_params=_tc_params,
    )(h, wa, wb)


def _edge_body(g_ref, e_ref, wc, b1, g, bt, w2, b2,
               enew_ref, enext_ref):
    pre = (g_ref[...]
           + jnp.dot(e_ref[...], wc[...], preferred_element_type=jnp.float32)
           + b1[...])
    r = jnp.maximum(pre, 0.0)
    ln = _ln(r, g[...], bt[...])
    en = jnp.dot(ln, w2[...], preferred_element_type=jnp.float32) + b2[...]
    enew_ref[...] = en
    enext_ref[...] = e_ref[...] + en


def _tc_edge(gsum, e, wc, b1, g, bt, w2, b2):
    return pl.pallas_call(
        _edge_body,
        grid=(E // BE,),
        in_specs=[_rows((BE, HID)), _rows((BE, HID)),
                  _full2((HID, HID)),
                  _full2((1, HID)), _full2((1, HID)), _full2((1, HID)),
                  _full2((HID, HID)), _full2((1, HID))],
        out_specs=[_rows((BE, HID)), _rows((BE, HID))],
        out_shape=(jax.ShapeDtypeStruct((E, HID), jnp.float32),
                   jax.ShapeDtypeStruct((E, HID), jnp.float32)),
        compiler_params=_tc_params,
    )(gsum, e, wc, b1.reshape(1, HID), g.reshape(1, HID),
      bt.reshape(1, HID), w2, b2.reshape(1, HID))


def _node_body(h_ref, s_ref, cnt_ref, wh, wa, b1, g, bt, w2, b2, out_ref):
    inv = 1.0 / jnp.maximum(cnt_ref[...][:, :1], 1.0)
    agg = s_ref[...] * inv
    pre = (jnp.dot(h_ref[...], wh[...], preferred_element_type=jnp.float32)
           + jnp.dot(agg, wa[...], preferred_element_type=jnp.float32)
           + b1[...])
    r = jnp.maximum(pre, 0.0)
    ln = _ln(r, g[...], bt[...])
    out_ref[...] = h_ref[...] + jnp.dot(
        ln, w2[...], preferred_element_type=jnp.float32) + b2[...]


def _tc_node(h, s, cnt, wh, wa, b1, g, bt, w2, b2):
    return pl.pallas_call(
        _node_body,
        grid=(N // BN,),
        in_specs=[_rows((BN, HID)), _rows((BN, HID)), _rows((BN, CNTW)),
                  _full2((HID, HID)), _full2((HID, HID)),
                  _full2((1, HID)), _full2((1, HID)), _full2((1, HID)),
                  _full2((HID, HID)), _full2((1, HID))],
        out_specs=_rows((BN, HID)),
        out_shape=jax.ShapeDtypeStruct((N, HID), jnp.float32),
        compiler_params=_tc_params,
    )(h, s, cnt, wh, wa, b1.reshape(1, HID), g.reshape(1, HID),
      bt.reshape(1, HID), w2, b2.reshape(1, HID))


def _enc_body(x_ref, w1, b1, w2, b2, out_ref):
    r = jnp.maximum(
        jnp.dot(x_ref[...], w1[...], preferred_element_type=jnp.float32)
        + b1[...], 0.0)
    out_ref[...] = jnp.dot(r, w2[...], preferred_element_type=jnp.float32) + b2[...]


def _tc_enc(x, w1, b1, w2, b2, blk_rows):
    rows, din = x.shape
    dout = w2.shape[1]
    return pl.pallas_call(
        _enc_body,
        grid=(rows // blk_rows,),
        in_specs=[_rows((blk_rows, din)), _full2((din, HID)), _full2((1, HID)),
                  _full2((HID, dout)), _full2((1, dout))],
        out_specs=_rows((blk_rows, dout)),
        out_shape=jax.ShapeDtypeStruct((rows, dout), jnp.float32),
        compiler_params=_tc_params,
    )(x, w1, b1.reshape(1, -1), w2, b2.reshape(1, -1))


def kernel(x, edge_index, edge_attr, params):
    row = edge_index[0]
    col = edge_index[1]

    loc0 = jnp.where(col < HALF, col, TRASH)
    loc1 = jnp.where(col >= HALF, col - HALF, TRASH)
    loc01 = jnp.concatenate([loc0, loc1]).astype(jnp.int32)

    ones = jnp.ones((SCH, CNTW), jnp.float32)
    zeros16 = jnp.zeros((ZROWS, CNTW), jnp.float32)
    zeros64 = jnp.zeros((ZROWS, HID), jnp.float32)
    cnt = _sc_count(loc01, ones, zeros16)

    dep = jnp.minimum(cnt[0, 0], 0.0)

    (wn1, bn1), (wn2, bn2) = params['enc_n']
    h = _tc_enc(x, wn1 + dep, bn1, wn2, bn2, BN)
    (we1, be1), (we2, be2) = params['enc_e']
    e = _tc_enc(edge_attr, we1 + dep, be1, we2, be2, BE)

    for blk in params['blocks']:
        ew1, eb1 = blk['edge']['l1']
        ew2, eb2 = blk['edge']['l2']
        nw1, nb1 = blk['node']['l1']
        nw2, nb2 = blk['node']['l2']

        ts, td = _tc_pair(h, ew1[:HID], ew1[HID:2 * HID])
        gsum = _sc_gather_add(ts, td, row, col)
        e_new, e = _tc_edge(gsum, e, ew1[2 * HID:],
                            eb1, blk['edge']['ln_g'], blk['edge']['ln_b'],
                            ew2, eb2)
        s = _sc_scatter(loc01, e_new, zeros64)
        h = _tc_node(h, s, cnt, nw1[:HID], nw1[HID:], nb1,
                     blk['node']['ln_g'], blk['node']['ln_b'], nw2, nb2)

    (dw1, db1), (dw2, db2) = params['dec']
    return _tc_enc(h, dw1, db1, dw2, db2, BN)


# speedup vs baseline: 1.9710x; 1.0421x over previous
import functools

import jax
import jax.numpy as jnp
from jax import lax
from jax.experimental import pallas as pl
from jax.experimental.pallas import tpu as pltpu
from jax.experimental.pallas import tpu_sc as plsc

N = 50000
E = 800000
HID = 64

NC = 2
NS = 16
NW = NC * NS

_mesh = plsc.VectorSubcoreMesh(core_axis_name="c", subcore_axis_name="s")

CH = 128
EPW = 24960
MAIN = EPW * NW
TAIL = (E - MAIN) // NW
TW = 2 * HID


@functools.partial(
    pl.kernel,
    out_type=jax.ShapeDtypeStruct((E, HID), jnp.float32),
    mesh=_mesh,
    scratch_types=(
        pltpu.VMEM((CH,), jnp.int32),
        pltpu.VMEM((CH,), jnp.int32),
        pltpu.VMEM((CH, HID), jnp.float32),
        pltpu.VMEM((CH, HID), jnp.float32),
        pltpu.VMEM((CH, HID), jnp.float32),
        pltpu.SemaphoreType.DMA,
        pltpu.SemaphoreType.DMA,
    ),
    compiler_params=pltpu.CompilerParams(use_tc_tiling_on_sc=False),
)
def _sc_gather_add(ts_hbm, td_hbm, row_hbm, col_hbm, g_out,
                   ia, ib, ba, bb, gv, sa, sb):
    wid = lax.axis_index("s") * NC + lax.axis_index("c")
    base = wid * EPW

    def chunk(off, n):
        pltpu.sync_copy(row_hbm.at[pl.ds(off, n)], ia.at[pl.ds(0, n)])
        pltpu.sync_copy(col_hbm.at[pl.ds(off, n)], ib.at[pl.ds(0, n)])
        ca = pltpu.async_copy(ts_hbm.at[ia.at[pl.ds(0, n)]],
                              ba.at[pl.ds(0, n)], sa)
        cb = pltpu.async_copy(td_hbm.at[ib.at[pl.ds(0, n)]],
                              bb.at[pl.ds(0, n)], sb)
        ca.wait()
        cb.wait()

        def addrow(k, carry):
            for j in range(HID // 16):
                gv[k, pl.ds(j * 16, 16)] = (
                    ba[k, pl.ds(j * 16, 16)]
                    + bb[k, pl.ds(j * 16, 16)])
            return carry

        lax.fori_loop(0, n, addrow, 0)
        pltpu.sync_copy(gv.at[pl.ds(0, n)], g_out.at[pl.ds(off, n)])

    def body(i, carry):
        chunk(base + i * CH, CH)
        return carry

    lax.fori_loop(0, EPW // CH, body, 0)
    chunk(MAIN + wid * TAIL, TAIL)


EIN = 16


@functools.partial(
    pl.kernel,
    out_type=jax.ShapeDtypeStruct((E, EIN), jnp.float32),
    mesh=_mesh,
    scratch_types=(
        pltpu.VMEM((CH,), jnp.int32),
        pltpu.VMEM((CH, EIN), jnp.float32),
        pltpu.SemaphoreType.DMA,
    ),
    compiler_params=pltpu.CompilerParams(use_tc_tiling_on_sc=False),
)
def _sc_permute(ea_hbm, perm_hbm, out_hbm, ia, buf, sem):
    wid = lax.axis_index("s") * NC + lax.axis_index("c")
    base = wid * EPW

    def chunk(off, n):
        pltpu.sync_copy(perm_hbm.at[pl.ds(off, n)], ia.at[pl.ds(0, n)])
        pltpu.async_copy(ea_hbm.at[ia.at[pl.ds(0, n)]],
                         buf.at[pl.ds(0, n)], sem).wait()
        pltpu.sync_copy(buf.at[pl.ds(0, n)], out_hbm.at[pl.ds(off, n)])

    def body(i, carry):
        chunk(base + i * CH, CH)
        return carry

    lax.fori_loop(0, EPW // CH, body, 0)
    chunk(MAIN + wid * TAIL, TAIL)


SCH = 128
NCHUNK = E // SCH
HALF = N // NC
ZROWS = 1568
ACC_ROWS = ZROWS * NS
TRASH = 25024
LASTZ = HALF - (NS - 1) * ZROWS


@functools.partial(
    pl.kernel,
    out_type=jax.ShapeDtypeStruct((N, HID), jnp.float32),
    mesh=_mesh,
    scratch_types=(
        pltpu.VMEM((16,), jnp.int32),
        pltpu.VMEM((SCH,), jnp.int32),
        pltpu.VMEM((SCH, HID), jnp.float32),
        pltpu.VMEM_SHARED((ACC_ROWS, HID), jnp.float32),
    ),
    compiler_params=pltpu.CompilerParams(use_tc_tiling_on_sc=False,
                                         needs_layout_passes=False),
)
def _sc_scatter(loc_hbm, val_hbm, zeros_hbm, splits_hbm, out_hbm,
                spv, il, vv, acc):
    cid = lax.axis_index("c")
    tid = lax.axis_index("s")
    nbase = cid * HALF
    ebase = cid * E

    pltpu.sync_copy(zeros_hbm, acc.at[pl.ds(tid * ZROWS, ZROWS)])
    pltpu.sync_copy(splits_hbm, spv)
    split = jnp.max(spv[...])
    k_lo = jnp.where(cid == 0, 0, split // SCH)
    k_hi = jnp.where(cid == 0, (split + SCH - 1) // SCH, NCHUNK)
    plsc.subcore_barrier()

    ntrip = jnp.maximum(k_hi - (k_lo + tid) + (NS - 1), 0) // NS

    def body(i, carry):
        off = (k_lo + tid + i * NS) * SCH
        pltpu.sync_copy(loc_hbm.at[pl.ds(ebase + off, SCH)], il)
        pltpu.sync_copy(val_hbm.at[pl.ds(off, SCH)], vv)
        pltpu.sync_copy(vv, acc.at[il], add=True)
        return carry

    lax.fori_loop(0, ntrip, body, 0)
    plsc.subcore_barrier()

    @pl.when(tid < NS - 1)
    def _():
        pltpu.sync_copy(acc.at[pl.ds(tid * ZROWS, ZROWS)],
                        out_hbm.at[pl.ds(nbase + tid * ZROWS, ZROWS)])

    @pl.when(tid == NS - 1)
    def _():
        pltpu.sync_copy(acc.at[pl.ds((NS - 1) * ZROWS, LASTZ)],
                        out_hbm.at[pl.ds(nbase + (NS - 1) * ZROWS, LASTZ)])


CNTW = 16


@functools.partial(
    pl.kernel,
    out_type=jax.ShapeDtypeStruct((N, CNTW), jnp.float32),
    mesh=_mesh,
    scratch_types=(
        pltpu.VMEM((16,), jnp.int32),
        pltpu.VMEM((SCH,), jnp.int32),
        pltpu.VMEM((SCH, CNTW), jnp.float32),
        pltpu.VMEM_SHARED((ACC_ROWS, CNTW), jnp.float32),
    ),
    compiler_params=pltpu.CompilerParams(use_tc_tiling_on_sc=False,
                                         needs_layout_passes=False),
)
def _sc_count(loc_hbm, ones_hbm, zeros_hbm, splits_hbm, out_hbm,
              spv, il, ones_v, acc):
    cid = lax.axis_index("c")
    tid = lax.axis_index("s")
    nbase = cid * HALF
    ebase = cid * E

    pltpu.sync_copy(zeros_hbm, acc.at[pl.ds(tid * ZROWS, ZROWS)])
    pltpu.sync_copy(ones_hbm, ones_v)
    pltpu.sync_copy(splits_hbm, spv)
    split = jnp.max(spv[...])
    k_lo = jnp.where(cid == 0, 0, split // SCH)
    k_hi = jnp.where(cid == 0, (split + SCH - 1) // SCH, NCHUNK)
    plsc.subcore_barrier()

    ntrip = jnp.maximum(k_hi - (k_lo + tid) + (NS - 1), 0) // NS

    def body(i, carry):
        off = (k_lo + tid + i * NS) * SCH
        pltpu.sync_copy(loc_hbm.at[pl.ds(ebase + off, SCH)], il)
        pltpu.sync_copy(ones_v, acc.at[il], add=True)
        return carry

    lax.fori_loop(0, ntrip, body, 0)
    plsc.subcore_barrier()

    @pl.when(tid < NS - 1)
    def _():
        pltpu.sync_copy(acc.at[pl.ds(tid * ZROWS, ZROWS)],
                        out_hbm.at[pl.ds(nbase + tid * ZROWS, ZROWS)])

    @pl.when(tid == NS - 1)
    def _():
        pltpu.sync_copy(acc.at[pl.ds((NS - 1) * ZROWS, LASTZ)],
                        out_hbm.at[pl.ds(nbase + (NS - 1) * ZROWS, LASTZ)])


BE = 2000
BN = 2000

_tc_params = pltpu.CompilerParams(dimension_semantics=("arbitrary",))


def _full2(shape):
    return pl.BlockSpec(shape, lambda i: (0, 0))


def _rows(shape):
    return pl.BlockSpec(shape, lambda i: (i, 0))


def _ln(r, g, b):
    m = jnp.mean(r, axis=-1, keepdims=True)
    v = jnp.mean((r - m) ** 2, axis=-1, keepdims=True)
    return (r - m) * lax.rsqrt(v + 1e-5) * g + b


def _pair_body(h_ref, wa, wb, ts_ref, td_ref):
    ts_ref[...] = jnp.dot(h_ref[...], wa[...], preferred_element_type=jnp.float32)
    td_ref[...] = jnp.dot(h_ref[...], wb[...], preferred_element_type=jnp.float32)


def _tc_pair(h, wa, wb):
    return pl.pallas_call(
        _pair_body,
        grid=(N // BN,),
        in_specs=[_rows((BN, HID)), _full2((HID, HID)), _full2((HID, HID))],
        out_specs=[_rows((BN, HID)), _rows((BN, HID))],
        out_shape=(jax.ShapeDtypeStruct((N, HID), jnp.float32),
                   jax.ShapeDtypeStruct((N, HID), jnp.float32)),
        compiler_params=_tc_params,
    )(h, wa, wb)


def _edge_body(g_ref, e_ref, wc, b1, g, bt, w2, b2,
               enew_ref, enext_ref):
    pre = (g_ref[...]
           + jnp.dot(e_ref[...], wc[...], preferred_element_type=jnp.float32)
           + b1[...])
    r = jnp.maximum(pre, 0.0)
    ln = _ln(r, g[...], bt[...])
    en = jnp.dot(ln, w2[...], preferred_element_type=jnp.float32) + b2[...]
    enew_ref[...] = en
    enext_ref[...] = e_ref[...] + en


def _tc_edge(gsum, e, wc, b1, g, bt, w2, b2):
    return pl.pallas_call(
        _edge_body,
        grid=(E // BE,),
        in_specs=[_rows((BE, HID)), _rows((BE, HID)),
                  _full2((HID, HID)),
                  _full2((1, HID)), _full2((1, HID)), _full2((1, HID)),
                  _full2((HID, HID)), _full2((1, HID))],
        out_specs=[_rows((BE, HID)), _rows((BE, HID))],
        out_shape=(jax.ShapeDtypeStruct((E, HID), jnp.float32),
                   jax.ShapeDtypeStruct((E, HID), jnp.float32)),
        compiler_params=_tc_params,
    )(gsum, e, wc, b1.reshape(1, HID), g.reshape(1, HID),
      bt.reshape(1, HID), w2, b2.reshape(1, HID))


def _node_body(h_ref, s_ref, cnt_ref, wh, wa, b1, g, bt, w2, b2, out_ref):
    inv = 1.0 / jnp.maximum(cnt_ref[...][:, :1], 1.0)
    agg = s_ref[...] * inv
    pre = (jnp.dot(h_ref[...], wh[...], preferred_element_type=jnp.float32)
           + jnp.dot(agg, wa[...], preferred_element_type=jnp.float32)
           + b1[...])
    r = jnp.maximum(pre, 0.0)
    ln = _ln(r, g[...], bt[...])
    out_ref[...] = h_ref[...] + jnp.dot(
        ln, w2[...], preferred_element_type=jnp.float32) + b2[...]


def _tc_node(h, s, cnt, wh, wa, b1, g, bt, w2, b2):
    return pl.pallas_call(
        _node_body,
        grid=(N // BN,),
        in_specs=[_rows((BN, HID)), _rows((BN, HID)), _rows((BN, CNTW)),
                  _full2((HID, HID)), _full2((HID, HID)),
                  _full2((1, HID)), _full2((1, HID)), _full2((1, HID)),
                  _full2((HID, HID)), _full2((1, HID))],
        out_specs=_rows((BN, HID)),
        out_shape=jax.ShapeDtypeStruct((N, HID), jnp.float32),
        compiler_params=_tc_params,
    )(h, s, cnt, wh, wa, b1.reshape(1, HID), g.reshape(1, HID),
      bt.reshape(1, HID), w2, b2.reshape(1, HID))


def _enc_body(x_ref, w1, b1, w2, b2, out_ref):
    r = jnp.maximum(
        jnp.dot(x_ref[...], w1[...], preferred_element_type=jnp.float32)
        + b1[...], 0.0)
    out_ref[...] = jnp.dot(r, w2[...], preferred_element_type=jnp.float32) + b2[...]


def _tc_enc(x, w1, b1, w2, b2, blk_rows):
    rows, din = x.shape
    dout = w2.shape[1]
    return pl.pallas_call(
        _enc_body,
        grid=(rows // blk_rows,),
        in_specs=[_rows((blk_rows, din)), _full2((din, HID)), _full2((1, HID)),
                  _full2((HID, dout)), _full2((1, dout))],
        out_specs=_rows((blk_rows, dout)),
        out_shape=jax.ShapeDtypeStruct((rows, dout), jnp.float32),
        compiler_params=_tc_params,
    )(x, w1, b1.reshape(1, -1), w2, b2.reshape(1, -1))


def kernel(x, edge_index, edge_attr, params):
    row = edge_index[0]
    col = edge_index[1]

    half_bit = (col >= HALF).astype(jnp.int32)
    perm2 = jnp.argsort(half_bit, stable=True).astype(jnp.int32)
    split = (E - jnp.sum(half_bit)).astype(jnp.int32)
    splits = jnp.full((16,), 1, jnp.int32) * split
    rowp = row[perm2]
    colp = col[perm2]
    loc0 = jnp.where(colp < HALF, colp, TRASH)
    loc1 = jnp.where(colp >= HALF, colp - HALF, TRASH)
    loc01 = jnp.concatenate([loc0, loc1]).astype(jnp.int32)

    eap = _sc_permute(edge_attr, perm2)

    ones = jnp.ones((SCH, CNTW), jnp.float32)
    zeros16 = jnp.zeros((ZROWS, CNTW), jnp.float32)
    zeros64 = jnp.zeros((ZROWS, HID), jnp.float32)

    depp = jnp.minimum(jnp.abs(eap[0, 0]), 0.0)
    cnt = _sc_count(loc01, ones, zeros16, splits + depp.astype(jnp.int32))
    dep = jnp.minimum(cnt[0, 0], 0.0)

    (wn1, bn1), (wn2, bn2) = params['enc_n']
    h = _tc_enc(x, wn1 + dep, bn1, wn2, bn2, BN)
    (we1, be1), (we2, be2) = params['enc_e']
    e = _tc_enc(eap, we1 + dep, be1, we2, be2, BE)

    for blk in params['blocks']:
        ew1, eb1 = blk['edge']['l1']
        ew2, eb2 = blk['edge']['l2']
        nw1, nb1 = blk['node']['l1']
        nw2, nb2 = blk['node']['l2']

        ts, td = _tc_pair(h, ew1[:HID], ew1[HID:2 * HID])
        gsum = _sc_gather_add(ts, td, rowp, colp)
        e_new, e = _tc_edge(gsum, e, ew1[2 * HID:],
                            eb1, blk['edge']['ln_g'], blk['edge']['ln_b'],
                            ew2, eb2)
        s = _sc_scatter(loc01, e_new, zeros64, splits)
        h = _tc_node(h, s, cnt, nw1[:HID], nw1[HID:], nb1,
                     blk['node']['ln_g'], blk['node']['ln_b'], nw2, nb2)

    (dw1, db1), (dw2, db2) = params['dec']
    return _tc_enc(h, dw1, db1, dw2, db2, BN)


# speedup vs baseline: 2.1398x; 1.0856x over previous
import functools

import jax
import jax.numpy as jnp
from jax import lax
from jax.experimental import pallas as pl
from jax.experimental.pallas import tpu as pltpu
from jax.experimental.pallas import tpu_sc as plsc

N = 50000
E = 800000
HID = 64

NC = 2
NS = 16
NW = NC * NS

_mesh = plsc.VectorSubcoreMesh(core_axis_name="c", subcore_axis_name="s")

CH = 128
EPW = 24960
MAIN = EPW * NW
TAIL = (E - MAIN) // NW
TW = 2 * HID


NPAIR = (EPW // CH) // 2


@functools.partial(
    pl.kernel,
    out_type=jax.ShapeDtypeStruct((E, HID), jnp.float32),
    mesh=_mesh,
    scratch_types=(
        pltpu.VMEM((CH,), jnp.int32), pltpu.VMEM((CH,), jnp.int32),
        pltpu.VMEM((CH,), jnp.int32), pltpu.VMEM((CH,), jnp.int32),
        pltpu.VMEM((CH, HID), jnp.float32), pltpu.VMEM((CH, HID), jnp.float32),
        pltpu.VMEM((CH, HID), jnp.float32), pltpu.VMEM((CH, HID), jnp.float32),
        pltpu.VMEM((CH, HID), jnp.float32),
        pltpu.SemaphoreType.DMA, pltpu.SemaphoreType.DMA,
        pltpu.SemaphoreType.DMA, pltpu.SemaphoreType.DMA,
    ),
    compiler_params=pltpu.CompilerParams(use_tc_tiling_on_sc=False),
)
def _sc_gather_add(ts_hbm, td_hbm, row_hbm, col_hbm, g_out,
                   ia0, ia1, ib0, ib1, ba0, ba1, bb0, bb1, gv,
                   sa0, sa1, sb0, sb1):
    wid = lax.axis_index("s") * NC + lax.axis_index("c")
    base = wid * EPW
    IA = (ia0, ia1)
    IB = (ib0, ib1)
    BA = (ba0, ba1)
    BB = (bb0, bb1)
    SA = (sa0, sa1)
    SB = (sb0, sb1)

    def fire(off, s):
        pltpu.sync_copy(row_hbm.at[pl.ds(off, CH)], IA[s])
        pltpu.sync_copy(col_hbm.at[pl.ds(off, CH)], IB[s])
        pltpu.async_copy(ts_hbm.at[IA[s]], BA[s], SA[s])
        pltpu.async_copy(td_hbm.at[IB[s]], BB[s], SB[s])

    def drain(off, s):
        pltpu.make_async_copy(ts_hbm.at[IA[s]], BA[s], SA[s]).wait()
        pltpu.make_async_copy(td_hbm.at[IB[s]], BB[s], SB[s]).wait()

        def addrow(k, carry):
            for j in range(HID // 16):
                gv[k, pl.ds(j * 16, 16)] = (
                    BA[s][k, pl.ds(j * 16, 16)]
                    + BB[s][k, pl.ds(j * 16, 16)])
            return carry

        lax.fori_loop(0, CH, addrow, 0)
        pltpu.sync_copy(gv, g_out.at[pl.ds(off, CH)])

    fire(base, 0)

    def body(i, carry):
        off = base + 2 * i * CH
        fire(off + CH, 1)
        drain(off, 0)
        fire(off + 2 * CH, 0)
        drain(off + CH, 1)
        return carry

    lax.fori_loop(0, NPAIR, body, 0)
    drain(base + 2 * NPAIR * CH, 0)

    offt = MAIN + wid * TAIL
    pltpu.sync_copy(row_hbm.at[pl.ds(offt, TAIL)], ia1.at[pl.ds(0, TAIL)])
    pltpu.sync_copy(col_hbm.at[pl.ds(offt, TAIL)], ib1.at[pl.ds(0, TAIL)])
    ca = pltpu.async_copy(ts_hbm.at[ia1.at[pl.ds(0, TAIL)]],
                          ba1.at[pl.ds(0, TAIL)], sa1)
    cb = pltpu.async_copy(td_hbm.at[ib1.at[pl.ds(0, TAIL)]],
                          bb1.at[pl.ds(0, TAIL)], sb1)
    ca.wait()
    cb.wait()

    def addrow_t(k, carry):
        for j in range(HID // 16):
            gv[k, pl.ds(j * 16, 16)] = (
                ba1[k, pl.ds(j * 16, 16)] + bb1[k, pl.ds(j * 16, 16)])
        return carry

    lax.fori_loop(0, TAIL, addrow_t, 0)
    pltpu.sync_copy(gv.at[pl.ds(0, TAIL)], g_out.at[pl.ds(offt, TAIL)])


EIN = 16


@functools.partial(
    pl.kernel,
    out_type=jax.ShapeDtypeStruct((E, EIN), jnp.float32),
    mesh=_mesh,
    scratch_types=(
        pltpu.VMEM((CH,), jnp.int32),
        pltpu.VMEM((CH, EIN), jnp.float32),
        pltpu.SemaphoreType.DMA,
    ),
    compiler_params=pltpu.CompilerParams(use_tc_tiling_on_sc=False),
)
def _sc_permute(ea_hbm, perm_hbm, out_hbm, ia, buf, sem):
    wid = lax.axis_index("s") * NC + lax.axis_index("c")
    base = wid * EPW

    def chunk(off, n):
        pltpu.sync_copy(perm_hbm.at[pl.ds(off, n)], ia.at[pl.ds(0, n)])
        pltpu.async_copy(ea_hbm.at[ia.at[pl.ds(0, n)]],
                         buf.at[pl.ds(0, n)], sem).wait()
        pltpu.sync_copy(buf.at[pl.ds(0, n)], out_hbm.at[pl.ds(off, n)])

    def body(i, carry):
        chunk(base + i * CH, CH)
        return carry

    lax.fori_loop(0, EPW // CH, body, 0)
    chunk(MAIN + wid * TAIL, TAIL)


SCH = 128
NCHUNK = E // SCH
HALF = N // NC
ZROWS = 1568
ACC_ROWS = ZROWS * NS
TRASH = 25024
LASTZ = HALF - (NS - 1) * ZROWS


@functools.partial(
    pl.kernel,
    out_type=jax.ShapeDtypeStruct((N, HID), jnp.float32),
    mesh=_mesh,
    scratch_types=(
        pltpu.VMEM((16,), jnp.int32),
        pltpu.VMEM((SCH,), jnp.int32),
        pltpu.VMEM((SCH, HID), jnp.float32),
        pltpu.VMEM_SHARED((ACC_ROWS, HID), jnp.float32),
    ),
    compiler_params=pltpu.CompilerParams(use_tc_tiling_on_sc=False,
                                         needs_layout_passes=False),
)
def _sc_scatter(loc_hbm, val_hbm, zeros_hbm, splits_hbm, out_hbm,
                spv, il, vv, acc):
    cid = lax.axis_index("c")
    tid = lax.axis_index("s")
    nbase = cid * HALF
    ebase = cid * E

    pltpu.sync_copy(zeros_hbm, acc.at[pl.ds(tid * ZROWS, ZROWS)])
    pltpu.sync_copy(splits_hbm, spv)
    split = jnp.max(spv[...])
    k_lo = jnp.where(cid == 0, 0, split // SCH)
    k_hi = jnp.where(cid == 0, (split + SCH - 1) // SCH, NCHUNK)
    plsc.subcore_barrier()

    ntrip = jnp.maximum(k_hi - (k_lo + tid) + (NS - 1), 0) // NS

    def body(i, carry):
        off = (k_lo + tid + i * NS) * SCH
        pltpu.sync_copy(loc_hbm.at[pl.ds(ebase + off, SCH)], il)
        pltpu.sync_copy(val_hbm.at[pl.ds(off, SCH)], vv)
        pltpu.sync_copy(vv, acc.at[il], add=True)
        return carry

    lax.fori_loop(0, ntrip, body, 0)
    plsc.subcore_barrier()

    @pl.when(tid < NS - 1)
    def _():
        pltpu.sync_copy(acc.at[pl.ds(tid * ZROWS, ZROWS)],
                        out_hbm.at[pl.ds(nbase + tid * ZROWS, ZROWS)])

    @pl.when(tid == NS - 1)
    def _():
        pltpu.sync_copy(acc.at[pl.ds((NS - 1) * ZROWS, LASTZ)],
                        out_hbm.at[pl.ds(nbase + (NS - 1) * ZROWS, LASTZ)])


CNTW = 16


@functools.partial(
    pl.kernel,
    out_type=jax.ShapeDtypeStruct((N, CNTW), jnp.float32),
    mesh=_mesh,
    scratch_types=(
        pltpu.VMEM((16,), jnp.int32),
        pltpu.VMEM((SCH,), jnp.int32),
        pltpu.VMEM((SCH, CNTW), jnp.float32),
        pltpu.VMEM_SHARED((ACC_ROWS, CNTW), jnp.float32),
    ),
    compiler_params=pltpu.CompilerParams(use_tc_tiling_on_sc=False,
                                         needs_layout_passes=False),
)
def _sc_count(loc_hbm, ones_hbm, zeros_hbm, splits_hbm, out_hbm,
              spv, il, ones_v, acc):
    cid = lax.axis_index("c")
    tid = lax.axis_index("s")
    nbase = cid * HALF
    ebase = cid * E

    pltpu.sync_copy(zeros_hbm, acc.at[pl.ds(tid * ZROWS, ZROWS)])
    pltpu.sync_copy(ones_hbm, ones_v)
    pltpu.sync_copy(splits_hbm, spv)
    split = jnp.max(spv[...])
    k_lo = jnp.where(cid == 0, 0, split // SCH)
    k_hi = jnp.where(cid == 0, (split + SCH - 1) // SCH, NCHUNK)
    plsc.subcore_barrier()

    ntrip = jnp.maximum(k_hi - (k_lo + tid) + (NS - 1), 0) // NS

    def body(i, carry):
        off = (k_lo + tid + i * NS) * SCH
        pltpu.sync_copy(loc_hbm.at[pl.ds(ebase + off, SCH)], il)
        pltpu.sync_copy(ones_v, acc.at[il], add=True)
        return carry

    lax.fori_loop(0, ntrip, body, 0)
    plsc.subcore_barrier()

    @pl.when(tid < NS - 1)
    def _():
        pltpu.sync_copy(acc.at[pl.ds(tid * ZROWS, ZROWS)],
                        out_hbm.at[pl.ds(nbase + tid * ZROWS, ZROWS)])

    @pl.when(tid == NS - 1)
    def _():
        pltpu.sync_copy(acc.at[pl.ds((NS - 1) * ZROWS, LASTZ)],
                        out_hbm.at[pl.ds(nbase + (NS - 1) * ZROWS, LASTZ)])


BE = 2000
BN = 2000

_tc_params = pltpu.CompilerParams(dimension_semantics=("arbitrary",))


def _full2(shape):
    return pl.BlockSpec(shape, lambda i: (0, 0))


def _rows(shape):
    return pl.BlockSpec(shape, lambda i: (i, 0))


def _ln(r, g, b):
    m = jnp.mean(r, axis=-1, keepdims=True)
    v = jnp.mean((r - m) ** 2, axis=-1, keepdims=True)
    return (r - m) * lax.rsqrt(v + 1e-5) * g + b


def _pair_body(h_ref, wa, wb, ts_ref, td_ref):
    ts_ref[...] = jnp.dot(h_ref[...], wa[...], preferred_element_type=jnp.float32)
    td_ref[...] = jnp.dot(h_ref[...], wb[...], preferred_element_type=jnp.float32)


def _tc_pair(h, wa, wb):
    return pl.pallas_call(
        _pair_body,
        grid=(N // BN,),
        in_specs=[_rows((BN, HID)), _full2((HID, HID)), _full2((HID, HID))],
        out_specs=[_rows((BN, HID)), _rows((BN, HID))],
        out_shape=(jax.ShapeDtypeStruct((N, HID), jnp.float32),
                   jax.ShapeDtypeStruct((N, HID), jnp.float32)),
        compiler_params=_tc_params,
    )(h, wa, wb)


def _edge_body(g_ref, e_ref, wc, b1, g, bt, w2, b2,
               enew_ref, enext_ref):
    pre = (g_ref[...]
           + jnp.dot(e_ref[...], wc[...], preferred_element_type=jnp.float32)
           + b1[...])
    r = jnp.maximum(pre, 0.0)
    ln = _ln(r, g[...], bt[...])
    en = jnp.dot(ln, w2[...], preferred_element_type=jnp.float32) + b2[...]
    enew_ref[...] = en
    enext_ref[...] = e_ref[...] + en


def _tc_edge(gsum, e, wc, b1, g, bt, w2, b2):
    return pl.pallas_call(
        _edge_body,
        grid=(E // BE,),
        in_specs=[_rows((BE, HID)), _rows((BE, HID)),
                  _full2((HID, HID)),
                  _full2((1, HID)), _full2((1, HID)), _full2((1, HID)),
                  _full2((HID, HID)), _full2((1, HID))],
        out_specs=[_rows((BE, HID)), _rows((BE, HID))],
        out_shape=(jax.ShapeDtypeStruct((E, HID), jnp.float32),
                   jax.ShapeDtypeStruct((E, HID), jnp.float32)),
        compiler_params=_tc_params,
    )(gsum, e, wc, b1.reshape(1, HID), g.reshape(1, HID),
      bt.reshape(1, HID), w2, b2.reshape(1, HID))


def _node_body(h_ref, s_ref, cnt_ref, wh, wa, b1, g, bt, w2, b2, out_ref):
    inv = 1.0 / jnp.maximum(cnt_ref[...][:, :1], 1.0)
    agg = s_ref[...] * inv
    pre = (jnp.dot(h_ref[...], wh[...], preferred_element_type=jnp.float32)
           + jnp.dot(agg, wa[...], preferred_element_type=jnp.float32)
           + b1[...])
    r = jnp.maximum(pre, 0.0)
    ln = _ln(r, g[...], bt[...])
    out_ref[...] = h_ref[...] + jnp.dot(
        ln, w2[...], preferred_element_type=jnp.float32) + b2[...]


def _tc_node(h, s, cnt, wh, wa, b1, g, bt, w2, b2):
    return pl.pallas_call(
        _node_body,
        grid=(N // BN,),
        in_specs=[_rows((BN, HID)), _rows((BN, HID)), _rows((BN, CNTW)),
                  _full2((HID, HID)), _full2((HID, HID)),
                  _full2((1, HID)), _full2((1, HID)), _full2((1, HID)),
                  _full2((HID, HID)), _full2((1, HID))],
        out_specs=_rows((BN, HID)),
        out_shape=jax.ShapeDtypeStruct((N, HID), jnp.float32),
        compiler_params=_tc_params,
    )(h, s, cnt, wh, wa, b1.reshape(1, HID), g.reshape(1, HID),
      bt.reshape(1, HID), w2, b2.reshape(1, HID))


def _enc_body(x_ref, w1, b1, w2, b2, out_ref):
    r = jnp.maximum(
        jnp.dot(x_ref[...], w1[...], preferred_element_type=jnp.float32)
        + b1[...], 0.0)
    out_ref[...] = jnp.dot(r, w2[...], preferred_element_type=jnp.float32) + b2[...]


def _tc_enc(x, w1, b1, w2, b2, blk_rows):
    rows, din = x.shape
    dout = w2.shape[1]
    return pl.pallas_call(
        _enc_body,
        grid=(rows // blk_rows,),
        in_specs=[_rows((blk_rows, din)), _full2((din, HID)), _full2((1, HID)),
                  _full2((HID, dout)), _full2((1, dout))],
        out_specs=_rows((blk_rows, dout)),
        out_shape=jax.ShapeDtypeStruct((rows, dout), jnp.float32),
        compiler_params=_tc_params,
    )(x, w1, b1.reshape(1, -1), w2, b2.reshape(1, -1))


def kernel(x, edge_index, edge_attr, params):
    row = edge_index[0]
    col = edge_index[1]

    half_bit = (col >= HALF).astype(jnp.int32)
    perm2 = jnp.argsort(half_bit, stable=True).astype(jnp.int32)
    split = (E - jnp.sum(half_bit)).astype(jnp.int32)
    splits = jnp.full((16,), 1, jnp.int32) * split
    rowp = row[perm2]
    colp = col[perm2]
    loc0 = jnp.where(colp < HALF, colp, TRASH)
    loc1 = jnp.where(colp >= HALF, colp - HALF, TRASH)
    loc01 = jnp.concatenate([loc0, loc1]).astype(jnp.int32)

    eap = _sc_permute(edge_attr, perm2)

    ones = jnp.ones((SCH, CNTW), jnp.float32)
    zeros16 = jnp.zeros((ZROWS, CNTW), jnp.float32)
    zeros64 = jnp.zeros((ZROWS, HID), jnp.float32)

    depp = jnp.minimum(jnp.abs(eap[0, 0]), 0.0)
    cnt = _sc_count(loc01, ones, zeros16, splits + depp.astype(jnp.int32))
    dep = jnp.minimum(cnt[0, 0], 0.0)

    (wn1, bn1), (wn2, bn2) = params['enc_n']
    h = _tc_enc(x, wn1 + dep, bn1, wn2, bn2, BN)
    (we1, be1), (we2, be2) = params['enc_e']
    e = _tc_enc(eap, we1 + dep, be1, we2, be2, BE)

    for blk in params['blocks']:
        ew1, eb1 = blk['edge']['l1']
        ew2, eb2 = blk['edge']['l2']
        nw1, nb1 = blk['node']['l1']
        nw2, nb2 = blk['node']['l2']

        ts, td = _tc_pair(h, ew1[:HID], ew1[HID:2 * HID])
        gsum = _sc_gather_add(ts, td, rowp, colp)
        e_new, e = _tc_edge(gsum, e, ew1[2 * HID:],
                            eb1, blk['edge']['ln_g'], blk['edge']['ln_b'],
                            ew2, eb2)
        s = _sc_scatter(loc01, e_new, zeros64, splits)
        h = _tc_node(h, s, cnt, nw1[:HID], nw1[HID:], nb1,
                     blk['node']['ln_g'], blk['node']['ln_b'], nw2, nb2)

    (dw1, db1), (dw2, db2) = params['dec']
    return _tc_enc(h, dw1, db1, dw2, db2, BN)


# speedup vs baseline: 2.3344x; 1.0909x over previous
import functools

import jax
import jax.numpy as jnp
from jax import lax
from jax.experimental import pallas as pl
from jax.experimental.pallas import tpu as pltpu
from jax.experimental.pallas import tpu_sc as plsc

N = 50000
E = 800000
HID = 64

NC = 2
NS = 16
NW = NC * NS

_mesh = plsc.VectorSubcoreMesh(core_axis_name="c", subcore_axis_name="s")

CH = 128
EPW = 24960
MAIN = EPW * NW
TAIL = (E - MAIN) // NW
TW = 2 * HID


NPAIR = (EPW // CH) // 2


@functools.partial(
    pl.kernel,
    out_type=jax.ShapeDtypeStruct((E, HID), jnp.float32),
    mesh=_mesh,
    scratch_types=(
        pltpu.VMEM((CH,), jnp.int32), pltpu.VMEM((CH,), jnp.int32),
        pltpu.VMEM((CH,), jnp.int32), pltpu.VMEM((CH,), jnp.int32),
        pltpu.VMEM((CH, HID), jnp.float32), pltpu.VMEM((CH, HID), jnp.float32),
        pltpu.VMEM((CH, HID), jnp.float32), pltpu.VMEM((CH, HID), jnp.float32),
        pltpu.VMEM((CH, HID), jnp.float32),
        pltpu.SemaphoreType.DMA, pltpu.SemaphoreType.DMA,
        pltpu.SemaphoreType.DMA, pltpu.SemaphoreType.DMA,
    ),
    compiler_params=pltpu.CompilerParams(use_tc_tiling_on_sc=False),
)
def _sc_gather_add(ts_hbm, td_hbm, row_hbm, col_hbm, g_out,
                   ia0, ia1, ib0, ib1, ba0, ba1, bb0, bb1, gv,
                   sa0, sa1, sb0, sb1):
    wid = lax.axis_index("s") * NC + lax.axis_index("c")
    base = wid * EPW
    IA = (ia0, ia1)
    IB = (ib0, ib1)
    BA = (ba0, ba1)
    BB = (bb0, bb1)
    SA = (sa0, sa1)
    SB = (sb0, sb1)

    def fire(off, s):
        pltpu.sync_copy(row_hbm.at[pl.ds(off, CH)], IA[s])
        pltpu.sync_copy(col_hbm.at[pl.ds(off, CH)], IB[s])
        pltpu.async_copy(ts_hbm.at[IA[s]], BA[s], SA[s])
        pltpu.async_copy(td_hbm.at[IB[s]], BB[s], SB[s])

    def drain(off, s):
        pltpu.make_async_copy(ts_hbm.at[IA[s]], BA[s], SA[s]).wait()
        pltpu.make_async_copy(td_hbm.at[IB[s]], BB[s], SB[s]).wait()

        def addrow(k, carry):
            for j in range(HID // 16):
                gv[k, pl.ds(j * 16, 16)] = (
                    BA[s][k, pl.ds(j * 16, 16)]
                    + BB[s][k, pl.ds(j * 16, 16)])
            return carry

        lax.fori_loop(0, CH, addrow, 0)
        pltpu.sync_copy(gv, g_out.at[pl.ds(off, CH)])

    fire(base, 0)

    def body(i, carry):
        off = base + 2 * i * CH
        fire(off + CH, 1)
        drain(off, 0)
        fire(off + 2 * CH, 0)
        drain(off + CH, 1)
        return carry

    lax.fori_loop(0, NPAIR, body, 0)
    drain(base + 2 * NPAIR * CH, 0)

    offt = MAIN + wid * TAIL
    pltpu.sync_copy(row_hbm.at[pl.ds(offt, TAIL)], ia1.at[pl.ds(0, TAIL)])
    pltpu.sync_copy(col_hbm.at[pl.ds(offt, TAIL)], ib1.at[pl.ds(0, TAIL)])
    ca = pltpu.async_copy(ts_hbm.at[ia1.at[pl.ds(0, TAIL)]],
                          ba1.at[pl.ds(0, TAIL)], sa1)
    cb = pltpu.async_copy(td_hbm.at[ib1.at[pl.ds(0, TAIL)]],
                          bb1.at[pl.ds(0, TAIL)], sb1)
    ca.wait()
    cb.wait()

    def addrow_t(k, carry):
        for j in range(HID // 16):
            gv[k, pl.ds(j * 16, 16)] = (
                ba1[k, pl.ds(j * 16, 16)] + bb1[k, pl.ds(j * 16, 16)])
        return carry

    lax.fori_loop(0, TAIL, addrow_t, 0)
    pltpu.sync_copy(gv.at[pl.ds(0, TAIL)], g_out.at[pl.ds(offt, TAIL)])


EIN = 16


@functools.partial(
    pl.kernel,
    out_type=jax.ShapeDtypeStruct((E, EIN), jnp.float32),
    mesh=_mesh,
    scratch_types=(
        pltpu.VMEM((CH,), jnp.int32),
        pltpu.VMEM((CH, EIN), jnp.float32),
        pltpu.SemaphoreType.DMA,
    ),
    compiler_params=pltpu.CompilerParams(use_tc_tiling_on_sc=False),
)
def _sc_permute(ea_hbm, perm_hbm, out_hbm, ia, buf, sem):
    wid = lax.axis_index("s") * NC + lax.axis_index("c")
    base = wid * EPW

    def chunk(off, n):
        pltpu.sync_copy(perm_hbm.at[pl.ds(off, n)], ia.at[pl.ds(0, n)])
        pltpu.async_copy(ea_hbm.at[ia.at[pl.ds(0, n)]],
                         buf.at[pl.ds(0, n)], sem).wait()
        pltpu.sync_copy(buf.at[pl.ds(0, n)], out_hbm.at[pl.ds(off, n)])

    def body(i, carry):
        chunk(base + i * CH, CH)
        return carry

    lax.fori_loop(0, EPW // CH, body, 0)
    chunk(MAIN + wid * TAIL, TAIL)


SCH = 128
NCHUNK = E // SCH
HALF = N // NC
ZROWS = 1568
ACC_ROWS = ZROWS * NS
TRASH = 25024
LASTZ = HALF - (NS - 1) * ZROWS


@functools.partial(
    pl.kernel,
    out_type=jax.ShapeDtypeStruct((N, HID), jnp.float32),
    mesh=_mesh,
    scratch_types=(
        pltpu.VMEM((16,), jnp.int32),
        pltpu.VMEM((SCH,), jnp.int32), pltpu.VMEM((SCH,), jnp.int32),
        pltpu.VMEM((SCH, HID), jnp.float32), pltpu.VMEM((SCH, HID), jnp.float32),
        pltpu.VMEM_SHARED((ACC_ROWS, HID), jnp.float32),
        pltpu.SemaphoreType.DMA, pltpu.SemaphoreType.DMA,
        pltpu.SemaphoreType.DMA, pltpu.SemaphoreType.DMA,
    ),
    compiler_params=pltpu.CompilerParams(use_tc_tiling_on_sc=False,
                                         needs_layout_passes=False),
)
def _sc_scatter(loc_hbm, val_hbm, zeros_hbm, splits_hbm, out_hbm,
                spv, il0, il1, vv0, vv1, acc, sl0, sl1, sv0, sv1):
    cid = lax.axis_index("c")
    tid = lax.axis_index("s")
    nbase = cid * HALF
    ebase = cid * E
    IL = (il0, il1)
    VV = (vv0, vv1)
    SL = (sl0, sl1)
    SV = (sv0, sv1)

    pltpu.sync_copy(zeros_hbm, acc.at[pl.ds(tid * ZROWS, ZROWS)])
    pltpu.sync_copy(splits_hbm, spv)
    split = jnp.max(spv[...])
    k_lo = jnp.where(cid == 0, 0, split // SCH)
    k_hi = jnp.where(cid == 0, (split + SCH - 1) // SCH, NCHUNK)
    plsc.subcore_barrier()

    k0 = k_lo + tid
    ntrip = jnp.maximum(k_hi - k0 + (NS - 1), 0) // NS

    def fire(k, s):
        @pl.when(k < k_hi)
        def _():
            off = k * SCH
            pltpu.async_copy(loc_hbm.at[pl.ds(ebase + off, SCH)], IL[s], SL[s])
            pltpu.async_copy(val_hbm.at[pl.ds(off, SCH)], VV[s], SV[s])

    def drain(k, s):
        @pl.when(k < k_hi)
        def _():
            off = k * SCH
            pltpu.make_async_copy(loc_hbm.at[pl.ds(ebase + off, SCH)],
                                  IL[s], SL[s]).wait()
            pltpu.make_async_copy(val_hbm.at[pl.ds(off, SCH)],
                                  VV[s], SV[s]).wait()
            pltpu.sync_copy(VV[s], acc.at[IL[s]], add=True)

    fire(k0, 0)

    def body(i, carry):
        ka = k0 + 2 * i * NS
        fire(ka + NS, 1)
        drain(ka, 0)
        fire(ka + 2 * NS, 0)
        drain(ka + NS, 1)
        return carry

    lax.fori_loop(0, (ntrip + 1) // 2, body, 0)
    plsc.subcore_barrier()

    @pl.when(tid < NS - 1)
    def _():
        pltpu.sync_copy(acc.at[pl.ds(tid * ZROWS, ZROWS)],
                        out_hbm.at[pl.ds(nbase + tid * ZROWS, ZROWS)])

    @pl.when(tid == NS - 1)
    def _():
        pltpu.sync_copy(acc.at[pl.ds((NS - 1) * ZROWS, LASTZ)],
                        out_hbm.at[pl.ds(nbase + (NS - 1) * ZROWS, LASTZ)])


CNTW = 16


@functools.partial(
    pl.kernel,
    out_type=jax.ShapeDtypeStruct((N, CNTW), jnp.float32),
    mesh=_mesh,
    scratch_types=(
        pltpu.VMEM((16,), jnp.int32),
        pltpu.VMEM((SCH,), jnp.int32),
        pltpu.VMEM((SCH, CNTW), jnp.float32),
        pltpu.VMEM_SHARED((ACC_ROWS, CNTW), jnp.float32),
    ),
    compiler_params=pltpu.CompilerParams(use_tc_tiling_on_sc=False,
                                         needs_layout_passes=False),
)
def _sc_count(loc_hbm, ones_hbm, zeros_hbm, splits_hbm, out_hbm,
              spv, il, ones_v, acc):
    cid = lax.axis_index("c")
    tid = lax.axis_index("s")
    nbase = cid * HALF
    ebase = cid * E

    pltpu.sync_copy(zeros_hbm, acc.at[pl.ds(tid * ZROWS, ZROWS)])
    pltpu.sync_copy(ones_hbm, ones_v)
    pltpu.sync_copy(splits_hbm, spv)
    split = jnp.max(spv[...])
    k_lo = jnp.where(cid == 0, 0, split // SCH)
    k_hi = jnp.where(cid == 0, (split + SCH - 1) // SCH, NCHUNK)
    plsc.subcore_barrier()

    ntrip = jnp.maximum(k_hi - (k_lo + tid) + (NS - 1), 0) // NS

    def body(i, carry):
        off = (k_lo + tid + i * NS) * SCH
        pltpu.sync_copy(loc_hbm.at[pl.ds(ebase + off, SCH)], il)
        pltpu.sync_copy(ones_v, acc.at[il], add=True)
        return carry

    lax.fori_loop(0, ntrip, body, 0)
    plsc.subcore_barrier()

    @pl.when(tid < NS - 1)
    def _():
        pltpu.sync_copy(acc.at[pl.ds(tid * ZROWS, ZROWS)],
                        out_hbm.at[pl.ds(nbase + tid * ZROWS, ZROWS)])

    @pl.when(tid == NS - 1)
    def _():
        pltpu.sync_copy(acc.at[pl.ds((NS - 1) * ZROWS, LASTZ)],
                        out_hbm.at[pl.ds(nbase + (NS - 1) * ZROWS, LASTZ)])


BE = 2000
BN = 2000

_tc_params = pltpu.CompilerParams(dimension_semantics=("arbitrary",))


def _full2(shape):
    return pl.BlockSpec(shape, lambda i: (0, 0))


def _rows(shape):
    return pl.BlockSpec(shape, lambda i: (i, 0))


def _ln(r, g, b):
    m = jnp.mean(r, axis=-1, keepdims=True)
    v = jnp.mean((r - m) ** 2, axis=-1, keepdims=True)
    return (r - m) * lax.rsqrt(v + 1e-5) * g + b


def _pair_body(h_ref, wa, wb, ts_ref, td_ref):
    ts_ref[...] = jnp.dot(h_ref[...], wa[...], preferred_element_type=jnp.float32)
    td_ref[...] = jnp.dot(h_ref[...], wb[...], preferred_element_type=jnp.float32)


def _tc_pair(h, wa, wb):
    return pl.pallas_call(
        _pair_body,
        grid=(N // BN,),
        in_specs=[_rows((BN, HID)), _full2((HID, HID)), _full2((HID, HID))],
        out_specs=[_rows((BN, HID)), _rows((BN, HID))],
        out_shape=(jax.ShapeDtypeStruct((N, HID), jnp.float32),
                   jax.ShapeDtypeStruct((N, HID), jnp.float32)),
        compiler_params=_tc_params,
    )(h, wa, wb)


def _edge_body(g_ref, e_ref, wc, b1, g, bt, w2, b2,
               enew_ref, enext_ref):
    pre = (g_ref[...]
           + jnp.dot(e_ref[...], wc[...], preferred_element_type=jnp.float32)
           + b1[...])
    r = jnp.maximum(pre, 0.0)
    ln = _ln(r, g[...], bt[...])
    en = jnp.dot(ln, w2[...], preferred_element_type=jnp.float32) + b2[...]
    enew_ref[...] = en
    enext_ref[...] = e_ref[...] + en


def _tc_edge(gsum, e, wc, b1, g, bt, w2, b2):
    return pl.pallas_call(
        _edge_body,
        grid=(E // BE,),
        in_specs=[_rows((BE, HID)), _rows((BE, HID)),
                  _full2((HID, HID)),
                  _full2((1, HID)), _full2((1, HID)), _full2((1, HID)),
                  _full2((HID, HID)), _full2((1, HID))],
        out_specs=[_rows((BE, HID)), _rows((BE, HID))],
        out_shape=(jax.ShapeDtypeStruct((E, HID), jnp.float32),
                   jax.ShapeDtypeStruct((E, HID), jnp.float32)),
        compiler_params=_tc_params,
    )(gsum, e, wc, b1.reshape(1, HID), g.reshape(1, HID),
      bt.reshape(1, HID), w2, b2.reshape(1, HID))


def _node_body(h_ref, s_ref, cnt_ref, wh, wa, b1, g, bt, w2, b2, out_ref):
    inv = 1.0 / jnp.maximum(cnt_ref[...][:, :1], 1.0)
    agg = s_ref[...] * inv
    pre = (jnp.dot(h_ref[...], wh[...], preferred_element_type=jnp.float32)
           + jnp.dot(agg, wa[...], preferred_element_type=jnp.float32)
           + b1[...])
    r = jnp.maximum(pre, 0.0)
    ln = _ln(r, g[...], bt[...])
    out_ref[...] = h_ref[...] + jnp.dot(
        ln, w2[...], preferred_element_type=jnp.float32) + b2[...]


def _tc_node(h, s, cnt, wh, wa, b1, g, bt, w2, b2):
    return pl.pallas_call(
        _node_body,
        grid=(N // BN,),
        in_specs=[_rows((BN, HID)), _rows((BN, HID)), _rows((BN, CNTW)),
                  _full2((HID, HID)), _full2((HID, HID)),
                  _full2((1, HID)), _full2((1, HID)), _full2((1, HID)),
                  _full2((HID, HID)), _full2((1, HID))],
        out_specs=_rows((BN, HID)),
        out_shape=jax.ShapeDtypeStruct((N, HID), jnp.float32),
        compiler_params=_tc_params,
    )(h, s, cnt, wh, wa, b1.reshape(1, HID), g.reshape(1, HID),
      bt.reshape(1, HID), w2, b2.reshape(1, HID))


def _enc_body(x_ref, w1, b1, w2, b2, out_ref):
    r = jnp.maximum(
        jnp.dot(x_ref[...], w1[...], preferred_element_type=jnp.float32)
        + b1[...], 0.0)
    out_ref[...] = jnp.dot(r, w2[...], preferred_element_type=jnp.float32) + b2[...]


def _tc_enc(x, w1, b1, w2, b2, blk_rows):
    rows, din = x.shape
    dout = w2.shape[1]
    return pl.pallas_call(
        _enc_body,
        grid=(rows // blk_rows,),
        in_specs=[_rows((blk_rows, din)), _full2((din, HID)), _full2((1, HID)),
                  _full2((HID, dout)), _full2((1, dout))],
        out_specs=_rows((blk_rows, dout)),
        out_shape=jax.ShapeDtypeStruct((rows, dout), jnp.float32),
        compiler_params=_tc_params,
    )(x, w1, b1.reshape(1, -1), w2, b2.reshape(1, -1))


def kernel(x, edge_index, edge_attr, params):
    row = edge_index[0]
    col = edge_index[1]

    half_bit = (col >= HALF).astype(jnp.int32)
    perm2 = jnp.argsort(half_bit, stable=True).astype(jnp.int32)
    split = (E - jnp.sum(half_bit)).astype(jnp.int32)
    splits = jnp.full((16,), 1, jnp.int32) * split
    rowp = row[perm2]
    colp = col[perm2]
    loc0 = jnp.where(colp < HALF, colp, TRASH)
    loc1 = jnp.where(colp >= HALF, colp - HALF, TRASH)
    loc01 = jnp.concatenate([loc0, loc1]).astype(jnp.int32)

    eap = _sc_permute(edge_attr, perm2)

    ones = jnp.ones((SCH, CNTW), jnp.float32)
    zeros16 = jnp.zeros((ZROWS, CNTW), jnp.float32)
    zeros64 = jnp.zeros((ZROWS, HID), jnp.float32)

    depp = jnp.minimum(jnp.abs(eap[0, 0]), 0.0)
    cnt = _sc_count(loc01, ones, zeros16, splits + depp.astype(jnp.int32))
    dep = jnp.minimum(cnt[0, 0], 0.0)

    (wn1, bn1), (wn2, bn2) = params['enc_n']
    h = _tc_enc(x, wn1 + dep, bn1, wn2, bn2, BN)
    (we1, be1), (we2, be2) = params['enc_e']
    e = _tc_enc(eap, we1 + dep, be1, we2, be2, BE)

    for blk in params['blocks']:
        ew1, eb1 = blk['edge']['l1']
        ew2, eb2 = blk['edge']['l2']
        nw1, nb1 = blk['node']['l1']
        nw2, nb2 = blk['node']['l2']

        ts, td = _tc_pair(h, ew1[:HID], ew1[HID:2 * HID])
        gsum = _sc_gather_add(ts, td, rowp, colp)
        e_new, e = _tc_edge(gsum, e, ew1[2 * HID:],
                            eb1, blk['edge']['ln_g'], blk['edge']['ln_b'],
                            ew2, eb2)
        s = _sc_scatter(loc01, e_new, zeros64, splits)
        h = _tc_node(h, s, cnt, nw1[:HID], nw1[HID:], nb1,
                     blk['node']['ln_g'], blk['node']['ln_b'], nw2, nb2)

    (dw1, db1), (dw2, db2) = params['dec']
    return _tc_enc(h, dw1, db1, dw2, db2, BN)
